# Initial kernel scaffold; baseline (speedup 1.0000x reference)
import jax
import jax.numpy as jnp
from jax.experimental import pallas as pl


def kernel(item_ids, table):
    raise NotImplementedError("write your pallas kernel here")


# speedup vs baseline: 1.3170x; 1.3170x over previous
import functools

import jax
import jax.numpy as jnp
from jax import lax
from jax.experimental import pallas as pl
from jax.experimental.pallas import tpu as pltpu
from jax.experimental.pallas import tpu_sc as plsc

_NC = 2
_NS = 16
_NW = _NC * _NS
_CHUNK = 128
_NBUF = 8


def _make_gather(n_chunks: int, emb_dim: int):
    mesh = plsc.VectorSubcoreMesh(core_axis_name="c", subcore_axis_name="s")

    @functools.partial(
        pl.kernel,
        out_type=jax.ShapeDtypeStruct((_NW, n_chunks, _CHUNK, emb_dim),
                                      jnp.float32),
        mesh=mesh,
        compiler_params=pltpu.CompilerParams(use_tc_tiling_on_sc=False),
        scratch_types=[
            pltpu.VMEM((n_chunks, _CHUNK), jnp.int32),
            pltpu.VMEM((_NBUF, _CHUNK, emb_dim), jnp.float32),
            pltpu.SemaphoreType.DMA,
        ],
    )
    def gather_kernel(ids_hbm, table_hbm, out_hbm, idx_v, rows_v, gsem):
        wid = lax.axis_index("s") * _NC + lax.axis_index("c")
        pltpu.sync_copy(ids_hbm.at[wid], idx_v)

        for b in range(_NBUF):
            pltpu.async_copy(table_hbm.at[idx_v.at[b]], rows_v.at[b], gsem)

        @pl.loop(0, n_chunks, step=_NBUF)
        def _(j0):
            for b in range(_NBUF):
                j = j0 + b
                pltpu.make_async_copy(table_hbm.at[idx_v.at[j]],
                                      rows_v.at[b], gsem).wait()
                pltpu.sync_copy(rows_v.at[b], out_hbm.at[wid, j])
                nxt = j + _NBUF

                @pl.when(nxt < n_chunks)
                def _():
                    pltpu.async_copy(table_hbm.at[idx_v.at[nxt]],
                                     rows_v.at[b], gsem)

    return gather_kernel


def kernel(item_ids, table):
    batch, hist = item_ids.shape
    _, emb_dim = table.shape
    total = batch * hist
    assert total % (_NW * _CHUNK) == 0
    n_chunks = total // (_NW * _CHUNK)
    ids = item_ids.reshape(_NW, n_chunks, _CHUNK)
    out = _make_gather(n_chunks, emb_dim)(ids, table)
    return out.reshape(batch, hist, emb_dim)
